# Initial kernel scaffold; baseline (speedup 1.0000x reference)
import jax
import jax.numpy as jnp
from jax.experimental import pallas as pl


def kernel(x, mods, embed, W_gamma, b_gamma, W_beta, b_beta):
    raise NotImplementedError("write your pallas kernel here")


# speedup vs baseline: 2.7036x; 2.7036x over previous
import functools

import jax
import jax.numpy as jnp
from jax import lax
from jax.experimental import pallas as pl
from jax.experimental.pallas import tpu as pltpu
from jax.experimental.pallas import tpu_sc as plsc

_NUM_CORES = 2
_NUM_SUBCORES = 16
_NW = _NUM_CORES * _NUM_SUBCORES
_LANES = 16

_CHUNK = 128


def _sc_gather(embed, idx):
    M = idx.shape[0]
    F = embed.shape[1]
    per_w = M // _NW
    n_chunks = per_w // _CHUNK
    mesh = plsc.VectorSubcoreMesh(core_axis_name="c", subcore_axis_name="s")

    @functools.partial(
        pl.kernel,
        mesh=mesh,
        out_type=jax.ShapeDtypeStruct((M, F), jnp.float32),
        compiler_params=pltpu.CompilerParams(use_tc_tiling_on_sc=False),
        scratch_types=[
            pltpu.VMEM((per_w,), jnp.int32),
            pltpu.VMEM((_CHUNK, F), jnp.float32),
            pltpu.SemaphoreType.DMA,
        ],
    )
    def gather_kernel(table_hbm, idx_hbm, out_hbm, idx_v, rows_v, sem):
        wid = lax.axis_index("s") * _NUM_CORES + lax.axis_index("c")
        base = wid * per_w
        pltpu.sync_copy(idx_hbm.at[pl.ds(base, per_w)], idx_v)

        def clamp_body(i, carry):
            sl = pl.ds(i * _LANES, _LANES)
            idx_v[sl] = jnp.maximum(idx_v[sl], 1)
            return carry

        lax.fori_loop(0, per_w // _LANES, clamp_body, 0, unroll=4)

        def gather_body(c, carry):
            idx_sl = idx_v.at[pl.ds(c * _CHUNK, _CHUNK)]
            pltpu.async_copy(table_hbm.at[idx_sl], rows_v, sem).wait()
            pltpu.sync_copy(rows_v, out_hbm.at[pl.ds(base + c * _CHUNK, _CHUNK)])
            return carry

        lax.fori_loop(0, n_chunks, gather_body, 0)

    return gather_kernel(embed, idx)


def _tc_film(e, x2, wg_t, wb_t, bg, bb, blk):
    M, F = e.shape
    D = x2.shape[1]

    def body(e_ref, x_ref, wg_ref, wb_ref, bg_ref, bb_ref, o_ref):
        e_blk = e_ref[...]
        g = jnp.dot(e_blk, wg_ref[...], preferred_element_type=jnp.float32)
        b = jnp.dot(e_blk, wb_ref[...], preferred_element_type=jnp.float32)
        o_ref[...] = (g + (1.0 + bg_ref[...])) * x_ref[...] + (b + bb_ref[...])

    return pl.pallas_call(
        body,
        grid=(M // blk,),
        in_specs=[
            pl.BlockSpec((blk, F), lambda i: (i, 0)),
            pl.BlockSpec((blk, D), lambda i: (i, 0)),
            pl.BlockSpec((F, D), lambda i: (0, 0)),
            pl.BlockSpec((F, D), lambda i: (0, 0)),
            pl.BlockSpec((1, D), lambda i: (0, 0)),
            pl.BlockSpec((1, D), lambda i: (0, 0)),
        ],
        out_specs=pl.BlockSpec((blk, D), lambda i: (i, 0)),
        out_shape=jax.ShapeDtypeStruct((M, D), jnp.float32),
    )(e, x2, wg_t, wb_t, bg, bb)


def kernel(x, mods, embed, W_gamma, b_gamma, W_beta, b_beta):
    B, N, D = x.shape
    F = embed.shape[1]
    M = B * N
    idx = mods.reshape(M).astype(jnp.int32)
    e = _sc_gather(embed, idx)
    out = _tc_film(
        e,
        x.reshape(M, D),
        W_gamma.T,
        W_beta.T,
        b_gamma.reshape(1, D),
        b_beta.reshape(1, D),
        blk=2048,
    )
    return out.reshape(B, N, D)


# speedup vs baseline: 2.9754x; 1.1005x over previous
import functools

import jax
import jax.numpy as jnp
from jax import lax
from jax.experimental import pallas as pl
from jax.experimental.pallas import tpu as pltpu
from jax.experimental.pallas import tpu_sc as plsc

_NUM_CORES = 2
_NUM_SUBCORES = 16
_NW = _NUM_CORES * _NUM_SUBCORES
_LANES = 16

_CHUNK = 128
_SUP_G = 5
_SUP = _SUP_G * _CHUNK


def _sc_gather(embed, idx):
    M = idx.shape[0]
    F = embed.shape[1]
    per_w = M // _NW
    n_sup = per_w // _SUP
    mesh = plsc.VectorSubcoreMesh(core_axis_name="c", subcore_axis_name="s")

    @functools.partial(
        pl.kernel,
        mesh=mesh,
        out_type=jax.ShapeDtypeStruct((M, F), jnp.float32),
        compiler_params=pltpu.CompilerParams(use_tc_tiling_on_sc=False),
        scratch_types=[
            pltpu.VMEM((per_w,), jnp.int32),
            pltpu.VMEM((2, _SUP, F), jnp.float32),
            pltpu.SemaphoreType.DMA,
            pltpu.SemaphoreType.DMA,
            pltpu.SemaphoreType.DMA,
            pltpu.SemaphoreType.DMA,
        ],
    )
    def gather_kernel(table_hbm, idx_hbm, out_hbm, idx_v, rows_v, gs0, gs1, ws0, ws1):
        wid = lax.axis_index("s") * _NUM_CORES + lax.axis_index("c")
        base = wid * per_w
        pltpu.sync_copy(idx_hbm.at[pl.ds(base, per_w)], idx_v)

        def clamp_body(i, carry):
            sl = pl.ds(i * _LANES, _LANES)
            idx_v[sl] = jnp.maximum(idx_v[sl], 1)
            return carry

        lax.fori_loop(0, per_w // _LANES, clamp_body, 0, unroll=4)

        def do_super(s, slot, gsem, wsem):
            row0 = s * _SUP

            @pl.when(s >= 2)
            def _():
                pltpu.make_async_copy(
                    rows_v.at[slot], out_hbm.at[pl.ds(base, _SUP)], wsem
                ).wait()

            descs = [
                pltpu.async_copy(
                    table_hbm.at[idx_v.at[pl.ds(row0 + j * _CHUNK, _CHUNK)]],
                    rows_v.at[slot, pl.ds(j * _CHUNK, _CHUNK)],
                    gsem,
                )
                for j in range(_SUP_G)
            ]
            for d in descs:
                d.wait()

            pltpu.async_copy(
                rows_v.at[slot], out_hbm.at[pl.ds(base + row0, _SUP)], wsem
            )

        def body(p, carry):
            do_super(2 * p, 0, gs0, ws0)
            do_super(2 * p + 1, 1, gs1, ws1)
            return carry

        lax.fori_loop(0, n_sup // 2, body, 0)
        pltpu.make_async_copy(rows_v.at[0], out_hbm.at[pl.ds(base, _SUP)], ws0).wait()
        pltpu.make_async_copy(rows_v.at[1], out_hbm.at[pl.ds(base, _SUP)], ws1).wait()

    return gather_kernel(embed, idx)


def _tc_film(e, x2, wg_t, wb_t, bg, bb, blk):
    M, F = e.shape
    D = x2.shape[1]

    def body(e_ref, x_ref, wg_ref, wb_ref, bg_ref, bb_ref, o_ref):
        e_blk = e_ref[...]
        g = jnp.dot(e_blk, wg_ref[...], preferred_element_type=jnp.float32)
        b = jnp.dot(e_blk, wb_ref[...], preferred_element_type=jnp.float32)
        o_ref[...] = (g + (1.0 + bg_ref[...])) * x_ref[...] + (b + bb_ref[...])

    return pl.pallas_call(
        body,
        grid=(M // blk,),
        in_specs=[
            pl.BlockSpec((blk, F), lambda i: (i, 0)),
            pl.BlockSpec((blk, D), lambda i: (i, 0)),
            pl.BlockSpec((F, D), lambda i: (0, 0)),
            pl.BlockSpec((F, D), lambda i: (0, 0)),
            pl.BlockSpec((1, D), lambda i: (0, 0)),
            pl.BlockSpec((1, D), lambda i: (0, 0)),
        ],
        out_specs=pl.BlockSpec((blk, D), lambda i: (i, 0)),
        out_shape=jax.ShapeDtypeStruct((M, D), jnp.float32),
    )(e, x2, wg_t, wb_t, bg, bb)


def kernel(x, mods, embed, W_gamma, b_gamma, W_beta, b_beta):
    B, N, D = x.shape
    F = embed.shape[1]
    M = B * N
    idx = mods.reshape(M).astype(jnp.int32)
    e = _sc_gather(embed, idx)
    out = _tc_film(
        e,
        x.reshape(M, D),
        W_gamma.T,
        W_beta.T,
        b_gamma.reshape(1, D),
        b_beta.reshape(1, D),
        blk=2048,
    )
    return out.reshape(B, N, D)
